# baseline (speedup 1.0000x reference)
import functools

import jax
import jax.numpy as jnp
from jax import lax
from jax.experimental import pallas as pl
from jax.experimental.pallas import tpu as pltpu
from jax.experimental.pallas import tpu_sc as plsc

N_ENT = 10000
N_REL = 200
N_EDGE = 320000
D = 128

NPAD = 10240
RPAD = 256
RTBL = 201
NC, NS = 2, 16
NW = NC * NS
EPW = 10240
EPAD = NW * EPW
KB = 80
NB2 = EPW // KB
NI = NB2 // 4


def _k1_body(relf_ref, ww_ref, wb_ref, aw_ref, ab_ref, wrel_ref,
             st_ref, relw_ref):
    relf = relf_ref[...]
    aw = aw_ref[...]
    a1 = aw[:, :D]
    a2 = aw[:, D:]
    ww = ww_ref[...]
    wb = wb_ref[...]
    v1 = jnp.dot(a1, ww, preferred_element_type=jnp.float32)
    v2 = jnp.dot(a2, ww, preferred_element_type=jnp.float32)
    c1 = jnp.sum(a1 * wb)
    c2 = jnp.sum(a2 * wb) + ab_ref[0]
    v12 = jnp.concatenate([v1, v2], axis=0)
    st = lax.dot_general(v12, relf, (((1,), (1,)), ((), ())),
                         preferred_element_type=jnp.float32)
    c12 = jnp.concatenate([c1.reshape(1, 1), c2.reshape(1, 1)], axis=0)
    st_ref[...] = st + c12
    relw_ref[...] = jnp.dot(relf, wrel_ref[...],
                            preferred_element_type=jnp.float32)


def _pass_a_body(rows_hbm, et_hbm, qt_hbm, st_hbm,
                 alpha_hbm, deg_hbm, sa_hbm,
                 rows_v, et_v, qt_v, alpha_v, st_v, deg_loc, sa_loc):
    cid = lax.axis_index("c")
    sid = lax.axis_index("s")
    wid = sid * NC + cid
    pltpu.sync_copy(rows_hbm.at[wid], rows_v)
    pltpu.sync_copy(et_hbm.at[wid], et_v)
    pltpu.sync_copy(qt_hbm.at[wid], qt_v)
    pltpu.sync_copy(st_hbm, st_v)

    zero16 = jnp.zeros((16,), jnp.float32)
    ones16 = jnp.ones((16,), jnp.float32)

    def zbody(i, _):
        deg_loc[pl.ds(i * 16, 16)] = zero16
        sa_loc[pl.ds(i * 16, 16)] = zero16
        return 0
    lax.fori_loop(0, NPAD // 16, zbody, 0)

    def body(i, _):
        off = i * 16
        r16 = rows_v[pl.ds(off, 16)]
        e16 = et_v[pl.ds(off, 16)]
        q16 = qt_v[pl.ds(off, 16)]
        sv = plsc.load_gather(st_v, [e16])
        tv = plsc.load_gather(st_v, [q16 + RPAD])
        x = sv + tv
        x = jnp.where(x >= 0.0, x, 0.01 * x)
        a = jnp.exp(x)
        alpha_v[pl.ds(off, 16)] = a
        plsc.addupdate_scatter(sa_loc, [r16], a)
        plsc.addupdate_scatter(deg_loc, [r16], ones16)
        return 0
    lax.fori_loop(0, EPW // 16, body, 0)

    pltpu.sync_copy(alpha_v, alpha_hbm.at[wid])
    pltpu.sync_copy(deg_loc, deg_hbm.at[wid])
    pltpu.sync_copy(sa_loc, sa_hbm.at[wid])


def _k3_body(deg_ref, sa_ref, dinv_ref, g_ref):
    deg = jnp.sum(deg_ref[...], axis=0)
    sa = jnp.sum(sa_ref[...], axis=0)
    dinv = jnp.where(deg > 0.0, lax.rsqrt(deg), 0.0)
    dinv_ref[...] = dinv
    g_ref[...] = jnp.where(sa > 0.0, dinv / sa, 0.0)


def _pass_c_body(rows_hbm, cols_hbm, et_hbm, alpha_hbm, dinv_hbm, g_hbm,
                 pk_hbm,
                 rows_v, cols_v, et_v, alpha_v, dinv_v, g_v, pkv):
    cid = lax.axis_index("c")
    sid = lax.axis_index("s")
    wid = sid * NC + cid
    pltpu.sync_copy(rows_hbm.at[wid], rows_v)
    pltpu.sync_copy(cols_hbm.at[wid], cols_v)
    pltpu.sync_copy(et_hbm.at[wid], et_v)
    pltpu.sync_copy(alpha_hbm.at[wid], alpha_v)
    pltpu.sync_copy(dinv_hbm, dinv_v)
    pltpu.sync_copy(g_hbm, g_v)

    def body(b2, _):
        for kc in range(KB // 16):
            off = b2 * KB + kc * 16
            sl16 = pl.ds(off, 16)
            r16 = rows_v[sl16]
            c16 = cols_v[sl16]
            coef = (alpha_v[sl16]
                    * plsc.load_gather(g_v, [r16])
                    * plsc.load_gather(dinv_v, [c16]))
            dsl = pl.ds(kc * 16, 16)
            pkv[b2, 0, dsl] = r16
            pkv[b2, 1, dsl] = c16
            pkv[b2, 2, dsl] = et_v[sl16]
            pkv[b2, 3, dsl] = plsc.bitcast(coef, jnp.int32)
        return 0
    lax.fori_loop(0, NB2, body, 0)
    pltpu.sync_copy(pkv, pk_hbm.at[wid])


def _pass_b_body(pk_hbm, rel_hbm, ent_hbm,
                 acc_hbm,
                 xb0, xb1, rb0, rb1, eb0, eb1, eb2, eb3,
                 acc_sh, gs0, gs1, rs0, rs1, es0, es1, es2, es3, ss0, ss1):
    cid = lax.axis_index("c")
    sid = lax.axis_index("s")
    wid = sid * NC + cid
    xb = [xb0, xb1]
    rb = [rb0, rb1]
    eb = [eb0, eb1, eb2, eb3]
    gs = [gs0, gs1]
    rs = [rs0, rs1]
    es = [es0, es1, es2, es3]
    ss = [ss0, ss1]

    zero16 = jnp.zeros((16,), jnp.float32)

    def zbody(i, _):
        for f in range(8):
            xb0[i, pl.ds(f * 16, 16)] = zero16
        return 0
    lax.fori_loop(0, KB, zbody, 0)
    for z in range(8):
        pltpu.sync_copy(xb0, acc_sh.at[pl.ds(sid * 640 + z * KB, KB)])
    plsc.subcore_barrier()

    three16 = jnp.full((16,), 3, jnp.int32)

    pltpu.sync_copy(pk_hbm.at[wid, 0], eb[0])
    pltpu.async_copy(pk_hbm.at[wid, 1], eb[1], es[1])
    pltpu.async_copy(ent_hbm.at[eb[0].at[1]], xb[0], gs[0])
    pltpu.async_copy(rel_hbm.at[eb[0].at[2]], rb[0], rs[0])

    def outer(i, _):
        for sub in range(4):
            g = i * 4 + sub
            k = sub % 2
            kn = 1 - k
            sn = (sub + 1) % 4

            def issue_next_gather():
                pltpu.make_async_copy(pk_hbm.at[wid, g + 1], eb[sn],
                                      es[sn]).wait()
                pltpu.async_copy(ent_hbm.at[eb[sn].at[1]], xb[kn], gs[kn])
                pltpu.async_copy(rel_hbm.at[eb[sn].at[2]], rb[kn], rs[kn])

            def wait_prev_scatter():
                pltpu.make_async_copy(xb[kn], acc_sh.at[eb[(sub + 3) % 4]
                                                        .at[0]], ss[kn]).wait()
            if sub == 0:
                pl.when(i >= 1)(wait_prev_scatter)
            else:
                wait_prev_scatter()
            if sub == 3:
                pl.when(i < NI - 1)(issue_next_gather)
            else:
                issue_next_gather()

            pltpu.make_async_copy(ent_hbm.at[eb[sub].at[1]], xb[k],
                                  gs[k]).wait()
            pltpu.make_async_copy(rel_hbm.at[eb[sub].at[2]], rb[k],
                                  rs[k]).wait()

            @plsc.parallel_loop(0, KB, step=1, unroll=4)
            def edge(j):
                jv = jnp.full((16,), j, jnp.int32)
                c16i = plsc.load_gather(eb[sub], [three16, jv])
                c16 = plsc.bitcast(c16i, jnp.float32)
                for f in range(8):
                    sl = pl.ds(f * 16, 16)
                    xb[k][j, sl] = xb[k][j, sl] * rb[k][j, sl] * c16

            pltpu.async_copy(xb[k], acc_sh.at[eb[sub].at[0]], ss[k],
                             add=True)

            def issue_next_edgedata():
                pltpu.async_copy(pk_hbm.at[wid, g + 2], eb[(sub + 2) % 4],
                                 es[(sub + 2) % 4])
            if sub >= 2:
                pl.when(i < NI - 1)(issue_next_edgedata)
            else:
                issue_next_edgedata()
        return 0
    lax.fori_loop(0, NI, outer, 0)
    pltpu.make_async_copy(xb[1], acc_sh.at[eb[3].at[0]], ss[1]).wait()
    plsc.subcore_barrier()
    pltpu.sync_copy(acc_sh.at[pl.ds(sid * 640, 640)],
                    acc_hbm.at[cid, pl.ds(sid * 640, 640)])


def _k5_body(acc_ref, ent_ref, lrel_ref, wout_ref, wloop_ref, bias_ref,
             gam_ref, bet_ref, out_ref):
    acc = acc_ref[0, :, :] + acc_ref[1, :, :]
    h = jnp.dot(acc, wout_ref[...], preferred_element_type=jnp.float32)
    lr = jnp.dot(ent_ref[...] * lrel_ref[...], wloop_ref[...],
                 preferred_element_type=jnp.float32)
    out = h * 0.5 + lr * 0.5 + bias_ref[...]
    mask = (lax.broadcasted_iota(jnp.int32, (NPAD, 1), 0)
            < N_ENT).astype(jnp.float32)
    om = out * mask
    mean = jnp.sum(om, axis=0, keepdims=True) / N_ENT
    var = jnp.sum(om * om, axis=0, keepdims=True) / N_ENT - mean * mean
    y = (out - mean) * lax.rsqrt(var + 1e-5) * gam_ref[...] + bet_ref[...]
    out_ref[...] = jnp.where(y >= 0.0, y, 0.01 * y)


@functools.cache
def _sc_kernels():
    sc_mesh = plsc.VectorSubcoreMesh(core_axis_name="c", subcore_axis_name="s",
                                     num_cores=NC, num_subcores=NS)
    params = pltpu.CompilerParams(needs_layout_passes=False)
    pass_a = pl.kernel(
        _pass_a_body,
        out_type=(
            jax.ShapeDtypeStruct((NW, EPW), jnp.float32),
            jax.ShapeDtypeStruct((NW, NPAD), jnp.float32),
            jax.ShapeDtypeStruct((NW, NPAD), jnp.float32),
        ),
        mesh=sc_mesh,
        compiler_params=params,
        scratch_types=[
            pltpu.VMEM((EPW,), jnp.int32),
            pltpu.VMEM((EPW,), jnp.int32),
            pltpu.VMEM((EPW,), jnp.int32),
            pltpu.VMEM((EPW,), jnp.float32),
            pltpu.VMEM((2 * RPAD,), jnp.float32),
            pltpu.VMEM((NPAD,), jnp.float32),
            pltpu.VMEM((NPAD,), jnp.float32),
        ],
    )

    pass_c = pl.kernel(
        _pass_c_body,
        out_type=jax.ShapeDtypeStruct((NW, NB2, 4, KB), jnp.int32),
        mesh=sc_mesh,
        compiler_params=params,
        scratch_types=[
            pltpu.VMEM((EPW,), jnp.int32),
            pltpu.VMEM((EPW,), jnp.int32),
            pltpu.VMEM((EPW,), jnp.int32),
            pltpu.VMEM((EPW,), jnp.float32),
            pltpu.VMEM((NPAD,), jnp.float32),
            pltpu.VMEM((NPAD,), jnp.float32),
            pltpu.VMEM((NB2, 4, KB), jnp.int32),
        ],
    )

    pass_b = pl.kernel(
        _pass_b_body,
        out_type=jax.ShapeDtypeStruct((NC, NPAD, D), jnp.float32),
        mesh=sc_mesh,
        compiler_params=params,
        scratch_types=[
            pltpu.VMEM((KB, D), jnp.float32),
            pltpu.VMEM((KB, D), jnp.float32),
            pltpu.VMEM((KB, D), jnp.float32),
            pltpu.VMEM((KB, D), jnp.float32),
            pltpu.VMEM((4, KB), jnp.int32),
            pltpu.VMEM((4, KB), jnp.int32),
            pltpu.VMEM((4, KB), jnp.int32),
            pltpu.VMEM((4, KB), jnp.int32),
            pltpu.VMEM_SHARED((NPAD, D), jnp.float32),
            pltpu.SemaphoreType.DMA,
            pltpu.SemaphoreType.DMA,
            pltpu.SemaphoreType.DMA,
            pltpu.SemaphoreType.DMA,
            pltpu.SemaphoreType.DMA,
            pltpu.SemaphoreType.DMA,
            pltpu.SemaphoreType.DMA,
            pltpu.SemaphoreType.DMA,
            pltpu.SemaphoreType.DMA,
            pltpu.SemaphoreType.DMA,
        ],
    )
    return pass_a, pass_c, pass_b


def kernel(edge_index, edge_type, query_type, entity_embed, rel_embed,
           w_loop, w_out, w_rel, loop_rel, W_weight, W_bias, a_weight,
           a_bias, bias, bn_gamma, bn_beta):
    f32 = jnp.float32
    row = edge_index[0].astype(jnp.int32)
    col = edge_index[1].astype(jnp.int32)
    et = edge_type.astype(jnp.int32)
    qt = query_type.astype(jnp.int32)

    epad = EPAD - N_EDGE
    row_p = jnp.concatenate([row, jnp.full((epad,), N_ENT, jnp.int32)])
    col_p = jnp.concatenate([col, jnp.zeros((epad,), jnp.int32)])
    et_p = jnp.concatenate([et, jnp.zeros((epad,), jnp.int32)])
    qt_p = jnp.concatenate([qt, jnp.zeros((epad,), jnp.int32)])
    rows2 = row_p.reshape(NW, EPW)
    cols2 = col_p.reshape(NW, EPW)
    et2 = et_p.reshape(NW, EPW)
    qt2 = qt_p.reshape(NW, EPW)

    rel_full = jnp.concatenate([rel_embed, loop_rel], axis=0)
    relf = jnp.pad(rel_full, ((0, RPAD - RTBL), (0, 0)))
    ent_pad = jnp.pad(entity_embed, ((0, NPAD - N_ENT), (0, 0)))

    st, rel_w = pl.pallas_call(
        _k1_body,
        out_shape=(
            jax.ShapeDtypeStruct((2, RPAD), f32),
            jax.ShapeDtypeStruct((RPAD, D), f32),
        ),
    )(relf, W_weight, W_bias.reshape(1, D), a_weight,
      a_bias.reshape(1,), w_rel)

    pass_a, pass_c, pass_b = _sc_kernels()
    alpha2, deg_parts, sa_parts = pass_a(rows2, et2, qt2, st.reshape(-1))

    dinv, g = pl.pallas_call(
        _k3_body,
        out_shape=(
            jax.ShapeDtypeStruct((NPAD // 128, 128), f32),
            jax.ShapeDtypeStruct((NPAD // 128, 128), f32),
        ),
    )(deg_parts.reshape(NW, NPAD // 128, 128),
      sa_parts.reshape(NW, NPAD // 128, 128))

    pk = pass_c(rows2, cols2, et2, alpha2, dinv.reshape(-1), g.reshape(-1))

    acc_parts = pass_b(pk, relf, ent_pad)

    node_out = pl.pallas_call(
        _k5_body,
        out_shape=jax.ShapeDtypeStruct((NPAD, D), f32),
    )(acc_parts, ent_pad, loop_rel, w_out, w_loop, bias.reshape(1, D),
      bn_gamma.reshape(1, D), bn_beta.reshape(1, D))

    return node_out[:N_ENT], rel_w[:N_REL]


# speedup vs baseline: 1.0749x; 1.0749x over previous
import functools

import jax
import jax.numpy as jnp
from jax import lax
from jax.experimental import pallas as pl
from jax.experimental.pallas import tpu as pltpu
from jax.experimental.pallas import tpu_sc as plsc

N_ENT = 10000
N_REL = 200
N_EDGE = 320000
D = 128

NPAD = 10240
RPAD = 256
RTBL = 201
NC, NS = 2, 16
NW = NC * NS
EPW = 10240
EPAD = NW * EPW
KB = 80
NB2 = EPW // KB
NI = NB2 // 4
CB0 = 80
CB1 = 2 * NB2 - CB0


def _k1_body(relf_ref, ww_ref, wb_ref, aw_ref, ab_ref, wrel_ref,
             st_ref, relw_ref):
    relf = relf_ref[...]
    aw = aw_ref[...]
    a1 = aw[:, :D]
    a2 = aw[:, D:]
    ww = ww_ref[...]
    wb = wb_ref[...]
    v1 = jnp.dot(a1, ww, preferred_element_type=jnp.float32)
    v2 = jnp.dot(a2, ww, preferred_element_type=jnp.float32)
    c1 = jnp.sum(a1 * wb)
    c2 = jnp.sum(a2 * wb) + ab_ref[0]
    v12 = jnp.concatenate([v1, v2], axis=0)
    st = lax.dot_general(v12, relf, (((1,), (1,)), ((), ())),
                         preferred_element_type=jnp.float32)
    c12 = jnp.concatenate([c1.reshape(1, 1), c2.reshape(1, 1)], axis=0)
    st_ref[...] = st + c12
    relw_ref[...] = jnp.dot(relf, wrel_ref[...],
                            preferred_element_type=jnp.float32)


def _pass_a_body(rows_hbm, et_hbm, qt_hbm, st_hbm,
                 alpha_hbm, deg_hbm, sa_hbm,
                 rows_v, et_v, qt_v, alpha_v, st_v, deg_loc, sa_loc):
    cid = lax.axis_index("c")
    sid = lax.axis_index("s")
    wid = sid * NC + cid
    pltpu.sync_copy(rows_hbm.at[wid], rows_v)
    pltpu.sync_copy(et_hbm.at[wid], et_v)
    pltpu.sync_copy(qt_hbm.at[wid], qt_v)
    pltpu.sync_copy(st_hbm, st_v)

    zero16 = jnp.zeros((16,), jnp.float32)
    ones16 = jnp.ones((16,), jnp.float32)

    def zbody(i, _):
        deg_loc[pl.ds(i * 16, 16)] = zero16
        sa_loc[pl.ds(i * 16, 16)] = zero16
        return 0
    lax.fori_loop(0, NPAD // 16, zbody, 0)

    def body(i, _):
        off = i * 16
        r16 = rows_v[pl.ds(off, 16)]
        e16 = et_v[pl.ds(off, 16)]
        q16 = qt_v[pl.ds(off, 16)]
        sv = plsc.load_gather(st_v, [e16])
        tv = plsc.load_gather(st_v, [q16 + RPAD])
        x = sv + tv
        x = jnp.where(x >= 0.0, x, 0.01 * x)
        a = jnp.exp(x)
        alpha_v[pl.ds(off, 16)] = a
        plsc.addupdate_scatter(sa_loc, [r16], a)
        plsc.addupdate_scatter(deg_loc, [r16], ones16)
        return 0
    lax.fori_loop(0, EPW // 16, body, 0)

    pltpu.sync_copy(alpha_v, alpha_hbm.at[wid])
    pltpu.sync_copy(deg_loc, deg_hbm.at[wid])
    pltpu.sync_copy(sa_loc, sa_hbm.at[wid])


def _k3_body(deg_ref, sa_ref, dinv_ref, g_ref):
    deg = jnp.sum(deg_ref[...], axis=0)
    sa = jnp.sum(sa_ref[...], axis=0)
    dinv = jnp.where(deg > 0.0, lax.rsqrt(deg), 0.0)
    dinv_ref[...] = dinv
    g_ref[...] = jnp.where(sa > 0.0, dinv / sa, 0.0)


def _pass_c_body(rows_hbm, cols_hbm, et_hbm, alpha_hbm, dinv_hbm, g_hbm,
                 pk_hbm,
                 rows_v, cols_v, et_v, alpha_v, dinv_v, g_v, pkv):
    cid = lax.axis_index("c")
    sid = lax.axis_index("s")
    wid = sid * NC + cid
    pltpu.sync_copy(rows_hbm.at[wid], rows_v)
    pltpu.sync_copy(cols_hbm.at[wid], cols_v)
    pltpu.sync_copy(et_hbm.at[wid], et_v)
    pltpu.sync_copy(alpha_hbm.at[wid], alpha_v)
    pltpu.sync_copy(dinv_hbm, dinv_v)
    pltpu.sync_copy(g_hbm, g_v)

    def body(b2, _):
        for kc in range(KB // 16):
            off = b2 * KB + kc * 16
            sl16 = pl.ds(off, 16)
            r16 = rows_v[sl16]
            c16 = cols_v[sl16]
            coef = (alpha_v[sl16]
                    * plsc.load_gather(g_v, [r16])
                    * plsc.load_gather(dinv_v, [c16]))
            dsl = pl.ds(kc * 16, 16)
            pkv[b2, 0, dsl] = r16
            pkv[b2, 1, dsl] = c16
            pkv[b2, 2, dsl] = et_v[sl16]
            pkv[b2, 3, dsl] = plsc.bitcast(coef, jnp.int32)
        return 0
    lax.fori_loop(0, NB2, body, 0)
    pltpu.sync_copy(pkv, pk_hbm.at[wid])


def _pass_b_body(pk_hbm, rel_hbm, ent_hbm,
                 acc_hbm,
                 rel_v, xb0, xb1, eb0, eb1, eb2, eb3,
                 acc_sh, gs0, gs1, es0, es1, es2, es3, ss0, ss1):
    cid = lax.axis_index("c")
    sid = lax.axis_index("s")
    wid = sid * NC + cid
    xb = [xb0, xb1]
    eb = [eb0, eb1, eb2, eb3]
    gs = [gs0, gs1]
    es = [es0, es1, es2, es3]
    ss = [ss0, ss1]
    pltpu.sync_copy(rel_hbm, rel_v)

    zero16 = jnp.zeros((16,), jnp.float32)

    def zbody(i, _):
        for f in range(8):
            xb0[i, pl.ds(f * 16, 16)] = zero16
        return 0
    lax.fori_loop(0, KB, zbody, 0)
    for z in range(8):
        pltpu.sync_copy(xb0, acc_sh.at[pl.ds(sid * 640 + z * KB, KB)])
    plsc.subcore_barrier()

    iota16 = lax.iota(jnp.int32, 16)
    two16 = jnp.full((16,), 2, jnp.int32)
    three16 = jnp.full((16,), 3, jnp.int32)
    fiota = [jnp.int32(f * 16) + iota16 for f in range(8)]

    nit = jnp.where(cid == 0, CB0 // 4, CB1 // 4)

    def gbi(g):
        own = wid * NB2 + g
        sib = (wid - 1) * NB2 + (g - (NB2 - CB0))
        return jnp.where((cid == 1) & (g >= NB2), sib, own)

    pltpu.sync_copy(pk_hbm.at[gbi(0)], eb[0])
    pltpu.async_copy(pk_hbm.at[gbi(1)], eb[1], es[1])
    pltpu.async_copy(ent_hbm.at[eb[0].at[1]], xb[0], gs[0])

    def outer(i, _):
        for sub in range(4):
            g = i * 4 + sub
            k = sub % 2
            kn = 1 - k
            sn = (sub + 1) % 4

            def issue_next_gather():
                pltpu.make_async_copy(pk_hbm.at[gbi(g + 1)], eb[sn],
                                      es[sn]).wait()
                pltpu.async_copy(ent_hbm.at[eb[sn].at[1]], xb[kn], gs[kn])

            def wait_prev_scatter():
                pltpu.make_async_copy(xb[kn], acc_sh.at[eb[(sub + 3) % 4]
                                                        .at[0]], ss[kn]).wait()
            if sub == 0:
                pl.when(i >= 1)(wait_prev_scatter)
            else:
                wait_prev_scatter()
            if sub == 3:
                pl.when(i < nit - 1)(issue_next_gather)
            else:
                issue_next_gather()

            pltpu.make_async_copy(ent_hbm.at[eb[sub].at[1]], xb[k],
                                  gs[k]).wait()

            @plsc.parallel_loop(0, KB, step=1, unroll=4)
            def edge(j):
                jv = jnp.full((16,), j, jnp.int32)
                e16 = plsc.load_gather(eb[sub], [two16, jv])
                c16i = plsc.load_gather(eb[sub], [three16, jv])
                c16 = plsc.bitcast(c16i, jnp.float32)
                base = e16 * D
                for f in range(8):
                    rv = plsc.load_gather(rel_v, [base + fiota[f]])
                    sl = pl.ds(f * 16, 16)
                    xb[k][j, sl] = xb[k][j, sl] * rv * c16

            pltpu.async_copy(xb[k], acc_sh.at[eb[sub].at[0]], ss[k],
                             add=True)

            def issue_next_edgedata():
                pltpu.async_copy(pk_hbm.at[gbi(g + 2)], eb[(sub + 2) % 4],
                                 es[(sub + 2) % 4])
            if sub >= 2:
                pl.when(i < nit - 1)(issue_next_edgedata)
            else:
                issue_next_edgedata()
        return 0
    lax.fori_loop(0, nit, outer, 0)
    pltpu.make_async_copy(xb[1], acc_sh.at[eb[3].at[0]], ss[1]).wait()
    plsc.subcore_barrier()
    pltpu.sync_copy(acc_sh.at[pl.ds(sid * 640, 640)],
                    acc_hbm.at[cid, pl.ds(sid * 640, 640)])


def _k5_body(acc_ref, ent_ref, lrel_ref, wout_ref, wloop_ref, bias_ref,
             gam_ref, bet_ref, out_ref):
    acc = acc_ref[0, :, :] + acc_ref[1, :, :]
    h = jnp.dot(acc, wout_ref[...], preferred_element_type=jnp.float32)
    lr = jnp.dot(ent_ref[...] * lrel_ref[...], wloop_ref[...],
                 preferred_element_type=jnp.float32)
    out = h * 0.5 + lr * 0.5 + bias_ref[...]
    mask = (lax.broadcasted_iota(jnp.int32, (NPAD, 1), 0)
            < N_ENT).astype(jnp.float32)
    om = out * mask
    mean = jnp.sum(om, axis=0, keepdims=True) / N_ENT
    var = jnp.sum(om * om, axis=0, keepdims=True) / N_ENT - mean * mean
    y = (out - mean) * lax.rsqrt(var + 1e-5) * gam_ref[...] + bet_ref[...]
    out_ref[...] = jnp.where(y >= 0.0, y, 0.01 * y)


@functools.cache
def _sc_kernels():
    sc_mesh = plsc.VectorSubcoreMesh(core_axis_name="c", subcore_axis_name="s",
                                     num_cores=NC, num_subcores=NS)
    params = pltpu.CompilerParams(needs_layout_passes=False)
    pass_a = pl.kernel(
        _pass_a_body,
        out_type=(
            jax.ShapeDtypeStruct((NW, EPW), jnp.float32),
            jax.ShapeDtypeStruct((NW, NPAD), jnp.float32),
            jax.ShapeDtypeStruct((NW, NPAD), jnp.float32),
        ),
        mesh=sc_mesh,
        compiler_params=params,
        scratch_types=[
            pltpu.VMEM((EPW,), jnp.int32),
            pltpu.VMEM((EPW,), jnp.int32),
            pltpu.VMEM((EPW,), jnp.int32),
            pltpu.VMEM((EPW,), jnp.float32),
            pltpu.VMEM((2 * RPAD,), jnp.float32),
            pltpu.VMEM((NPAD,), jnp.float32),
            pltpu.VMEM((NPAD,), jnp.float32),
        ],
    )

    pass_c = pl.kernel(
        _pass_c_body,
        out_type=jax.ShapeDtypeStruct((NW, NB2, 4, KB), jnp.int32),
        mesh=sc_mesh,
        compiler_params=params,
        scratch_types=[
            pltpu.VMEM((EPW,), jnp.int32),
            pltpu.VMEM((EPW,), jnp.int32),
            pltpu.VMEM((EPW,), jnp.int32),
            pltpu.VMEM((EPW,), jnp.float32),
            pltpu.VMEM((NPAD,), jnp.float32),
            pltpu.VMEM((NPAD,), jnp.float32),
            pltpu.VMEM((NB2, 4, KB), jnp.int32),
        ],
    )

    pass_b = pl.kernel(
        _pass_b_body,
        out_type=jax.ShapeDtypeStruct((NC, NPAD, D), jnp.float32),
        mesh=sc_mesh,
        compiler_params=params,
        scratch_types=[
            pltpu.VMEM((RTBL * D,), jnp.float32),
            pltpu.VMEM((KB, D), jnp.float32),
            pltpu.VMEM((KB, D), jnp.float32),
            pltpu.VMEM((4, KB), jnp.int32),
            pltpu.VMEM((4, KB), jnp.int32),
            pltpu.VMEM((4, KB), jnp.int32),
            pltpu.VMEM((4, KB), jnp.int32),
            pltpu.VMEM_SHARED((NPAD, D), jnp.float32),
            pltpu.SemaphoreType.DMA,
            pltpu.SemaphoreType.DMA,
            pltpu.SemaphoreType.DMA,
            pltpu.SemaphoreType.DMA,
            pltpu.SemaphoreType.DMA,
            pltpu.SemaphoreType.DMA,
            pltpu.SemaphoreType.DMA,
            pltpu.SemaphoreType.DMA,
        ],
    )
    return pass_a, pass_c, pass_b


def kernel(edge_index, edge_type, query_type, entity_embed, rel_embed,
           w_loop, w_out, w_rel, loop_rel, W_weight, W_bias, a_weight,
           a_bias, bias, bn_gamma, bn_beta):
    f32 = jnp.float32
    row = edge_index[0].astype(jnp.int32)
    col = edge_index[1].astype(jnp.int32)
    et = edge_type.astype(jnp.int32)
    qt = query_type.astype(jnp.int32)

    epad = EPAD - N_EDGE
    row_p = jnp.concatenate([row, jnp.full((epad,), N_ENT, jnp.int32)])
    col_p = jnp.concatenate([col, jnp.zeros((epad,), jnp.int32)])
    et_p = jnp.concatenate([et, jnp.zeros((epad,), jnp.int32)])
    qt_p = jnp.concatenate([qt, jnp.zeros((epad,), jnp.int32)])
    rows2 = row_p.reshape(NW, EPW)
    cols2 = col_p.reshape(NW, EPW)
    et2 = et_p.reshape(NW, EPW)
    qt2 = qt_p.reshape(NW, EPW)

    rel_full = jnp.concatenate([rel_embed, loop_rel], axis=0)
    relf = jnp.pad(rel_full, ((0, RPAD - RTBL), (0, 0)))
    ent_pad = jnp.pad(entity_embed, ((0, NPAD - N_ENT), (0, 0)))

    st, rel_w = pl.pallas_call(
        _k1_body,
        out_shape=(
            jax.ShapeDtypeStruct((2, RPAD), f32),
            jax.ShapeDtypeStruct((RPAD, D), f32),
        ),
    )(relf, W_weight, W_bias.reshape(1, D), a_weight,
      a_bias.reshape(1,), w_rel)

    pass_a, pass_c, pass_b = _sc_kernels()
    alpha2, deg_parts, sa_parts = pass_a(rows2, et2, qt2, st.reshape(-1))

    dinv, g = pl.pallas_call(
        _k3_body,
        out_shape=(
            jax.ShapeDtypeStruct((NPAD // 128, 128), f32),
            jax.ShapeDtypeStruct((NPAD // 128, 128), f32),
        ),
    )(deg_parts.reshape(NW, NPAD // 128, 128),
      sa_parts.reshape(NW, NPAD // 128, 128))

    pk = pass_c(rows2, cols2, et2, alpha2, dinv.reshape(-1), g.reshape(-1))

    acc_parts = pass_b(pk.reshape(NW * NB2, 4, KB), relf[:RTBL].reshape(-1),
                       ent_pad)

    node_out = pl.pallas_call(
        _k5_body,
        out_shape=jax.ShapeDtypeStruct((NPAD, D), f32),
    )(acc_parts, ent_pad, loop_rel, w_out, w_loop, bias.reshape(1, D),
      bn_gamma.reshape(1, D), bn_beta.reshape(1, D))

    return node_out[:N_ENT], rel_w[:N_REL]


# speedup vs baseline: 1.1958x; 1.1126x over previous
import functools

import jax
import jax.numpy as jnp
from jax import lax
from jax.experimental import pallas as pl
from jax.experimental.pallas import tpu as pltpu
from jax.experimental.pallas import tpu_sc as plsc

N_ENT = 10000
N_REL = 200
N_EDGE = 320000
D = 128

NPAD = 10240
RPAD = 256
RTBL = 201
NC, NS = 2, 16
NW = NC * NS
EPW = 10240
EPAD = NW * EPW
KB = 80
NB2 = EPW // KB
NI = NB2 // 4
FAST = 0
CBS = 80
CBF = 2 * NB2 - CBS


def _k1_body(relf_ref, ww_ref, wb_ref, aw_ref, ab_ref, wrel_ref,
             st_ref, relw_ref):
    relf = relf_ref[...]
    aw = aw_ref[...]
    a1 = aw[:, :D]
    a2 = aw[:, D:]
    ww = ww_ref[...]
    wb = wb_ref[...]
    v1 = jnp.dot(a1, ww, preferred_element_type=jnp.float32)
    v2 = jnp.dot(a2, ww, preferred_element_type=jnp.float32)
    c1 = jnp.sum(a1 * wb)
    c2 = jnp.sum(a2 * wb) + ab_ref[0]
    v12 = jnp.concatenate([v1, v2], axis=0)
    st = lax.dot_general(v12, relf, (((1,), (1,)), ((), ())),
                         preferred_element_type=jnp.float32)
    c12 = jnp.concatenate([c1.reshape(1, 1), c2.reshape(1, 1)], axis=0)
    st_ref[...] = st + c12
    relw_ref[...] = jnp.dot(relf, wrel_ref[...],
                            preferred_element_type=jnp.float32)


def _pass_a_body(rows_hbm, et_hbm, qt_hbm, st_hbm,
                 alpha_hbm, deg_hbm, sa_hbm,
                 rows_v, et_v, qt_v, alpha_v, st_v, deg_loc, sa_loc):
    cid = lax.axis_index("c")
    sid = lax.axis_index("s")
    wid = sid * NC + cid
    pltpu.sync_copy(rows_hbm.at[wid], rows_v)
    pltpu.sync_copy(et_hbm.at[wid], et_v)
    pltpu.sync_copy(qt_hbm.at[wid], qt_v)
    pltpu.sync_copy(st_hbm, st_v)

    zero16 = jnp.zeros((16,), jnp.float32)
    ones16 = jnp.ones((16,), jnp.float32)

    def zbody(i, _):
        deg_loc[pl.ds(i * 16, 16)] = zero16
        sa_loc[pl.ds(i * 16, 16)] = zero16
        return 0
    lax.fori_loop(0, NPAD // 16, zbody, 0)

    def body(i, _):
        off = i * 16
        r16 = rows_v[pl.ds(off, 16)]
        e16 = et_v[pl.ds(off, 16)]
        q16 = qt_v[pl.ds(off, 16)]
        sv = plsc.load_gather(st_v, [e16])
        tv = plsc.load_gather(st_v, [q16 + RPAD])
        x = sv + tv
        x = jnp.where(x >= 0.0, x, 0.01 * x)
        a = jnp.exp(x)
        alpha_v[pl.ds(off, 16)] = a
        plsc.addupdate_scatter(sa_loc, [r16], a)
        plsc.addupdate_scatter(deg_loc, [r16], ones16)
        return 0
    lax.fori_loop(0, EPW // 16, body, 0)

    pltpu.sync_copy(alpha_v, alpha_hbm.at[wid])
    pltpu.sync_copy(deg_loc, deg_hbm.at[wid])
    pltpu.sync_copy(sa_loc, sa_hbm.at[wid])


def _k3_body(deg_ref, sa_ref, dinv_ref, g_ref):
    deg = jnp.sum(deg_ref[...], axis=0)
    sa = jnp.sum(sa_ref[...], axis=0)
    dinv = jnp.where(deg > 0.0, lax.rsqrt(deg), 0.0)
    dinv_ref[...] = dinv
    g_ref[...] = jnp.where(sa > 0.0, dinv / sa, 0.0)


def _pass_c_body(rows_hbm, cols_hbm, et_hbm, alpha_hbm, dinv_hbm, g_hbm,
                 pk_hbm,
                 rows_v, cols_v, et_v, alpha_v, dinv_v, g_v, pkv):
    cid = lax.axis_index("c")
    sid = lax.axis_index("s")
    wid = sid * NC + cid
    pltpu.sync_copy(rows_hbm.at[wid], rows_v)
    pltpu.sync_copy(cols_hbm.at[wid], cols_v)
    pltpu.sync_copy(et_hbm.at[wid], et_v)
    pltpu.sync_copy(alpha_hbm.at[wid], alpha_v)
    pltpu.sync_copy(dinv_hbm, dinv_v)
    pltpu.sync_copy(g_hbm, g_v)

    def body(b2, _):
        for kc in range(KB // 16):
            off = b2 * KB + kc * 16
            sl16 = pl.ds(off, 16)
            r16 = rows_v[sl16]
            c16 = cols_v[sl16]
            coef = (alpha_v[sl16]
                    * plsc.load_gather(g_v, [r16])
                    * plsc.load_gather(dinv_v, [c16]))
            dsl = pl.ds(kc * 16, 16)
            pkv[b2, 0, dsl] = r16
            pkv[b2, 1, dsl] = c16
            pkv[b2, 2, dsl] = et_v[sl16]
            pkv[b2, 3, dsl] = plsc.bitcast(coef, jnp.int32)
        return 0
    lax.fori_loop(0, NB2, body, 0)
    pltpu.sync_copy(pkv, pk_hbm.at[wid])


def _pass_b_body(pk_hbm, rel_hbm, ent_hbm,
                 acc_hbm,
                 rel_v, xb0, xb1, eb0, eb1, eb2, eb3,
                 acc_sh, gs0, gs1, es0, es1, es2, es3, ss0, ss1):
    cid = lax.axis_index("c")
    sid = lax.axis_index("s")
    wid = sid * NC + cid
    xb = [xb0, xb1]
    eb = [eb0, eb1, eb2, eb3]
    gs = [gs0, gs1]
    es = [es0, es1, es2, es3]
    ss = [ss0, ss1]
    pltpu.sync_copy(rel_hbm, rel_v)

    zero16 = jnp.zeros((16,), jnp.float32)

    def zbody(i, _):
        for f in range(8):
            xb0[i, pl.ds(f * 16, 16)] = zero16
        return 0
    lax.fori_loop(0, KB, zbody, 0)
    for z in range(8):
        pltpu.sync_copy(xb0, acc_sh.at[pl.ds(sid * 640 + z * KB, KB)])
    plsc.subcore_barrier()

    iota16 = lax.iota(jnp.int32, 16)
    two16 = jnp.full((16,), 2, jnp.int32)
    three16 = jnp.full((16,), 3, jnp.int32)
    fiota = [jnp.int32(f * 16) + iota16 for f in range(8)]

    nit = jnp.where(cid == FAST, CBF // 4, CBS // 4)
    sib_off = 1 if FAST == 0 else -1

    def gbi(g):
        own = wid * NB2 + g
        sib = (wid + sib_off) * NB2 + (g - (NB2 - CBS))
        return jnp.where((cid == FAST) & (g >= NB2), sib, own)

    pltpu.sync_copy(pk_hbm.at[gbi(0)], eb[0])
    pltpu.async_copy(pk_hbm.at[gbi(1)], eb[1], es[1])
    pltpu.async_copy(ent_hbm.at[eb[0].at[1]], xb[0], gs[0])

    def outer(i, _):
        for sub in range(4):
            g = i * 4 + sub
            k = sub % 2
            kn = 1 - k
            sn = (sub + 1) % 4

            def issue_next_gather():
                pltpu.make_async_copy(pk_hbm.at[gbi(g + 1)], eb[sn],
                                      es[sn]).wait()
                pltpu.async_copy(ent_hbm.at[eb[sn].at[1]], xb[kn], gs[kn])

            def wait_prev_scatter():
                pltpu.make_async_copy(xb[kn], acc_sh.at[eb[(sub + 3) % 4]
                                                        .at[0]], ss[kn]).wait()
            if sub == 0:
                pl.when(i >= 1)(wait_prev_scatter)
            else:
                wait_prev_scatter()
            if sub == 3:
                pl.when(i < nit - 1)(issue_next_gather)
            else:
                issue_next_gather()

            pltpu.make_async_copy(ent_hbm.at[eb[sub].at[1]], xb[k],
                                  gs[k]).wait()

            @plsc.parallel_loop(0, KB, step=1, unroll=4)
            def edge(j):
                jv = jnp.full((16,), j, jnp.int32)
                e16 = plsc.load_gather(eb[sub], [two16, jv])
                c16i = plsc.load_gather(eb[sub], [three16, jv])
                c16 = plsc.bitcast(c16i, jnp.float32)
                base = e16 * D
                for f in range(8):
                    rv = plsc.load_gather(rel_v, [base + fiota[f]])
                    sl = pl.ds(f * 16, 16)
                    xb[k][j, sl] = xb[k][j, sl] * rv * c16

            pltpu.async_copy(xb[k], acc_sh.at[eb[sub].at[0]], ss[k],
                             add=True)

            def issue_next_edgedata():
                pltpu.async_copy(pk_hbm.at[gbi(g + 2)], eb[(sub + 2) % 4],
                                 es[(sub + 2) % 4])
            if sub >= 2:
                pl.when(i < nit - 1)(issue_next_edgedata)
            else:
                issue_next_edgedata()
        return 0
    lax.fori_loop(0, nit, outer, 0)
    pltpu.make_async_copy(xb[1], acc_sh.at[eb[3].at[0]], ss[1]).wait()
    plsc.subcore_barrier()
    pltpu.sync_copy(acc_sh.at[pl.ds(sid * 640, 640)],
                    acc_hbm.at[cid, pl.ds(sid * 640, 640)])


def _k5_body(acc_ref, ent_ref, lrel_ref, wout_ref, wloop_ref, bias_ref,
             gam_ref, bet_ref, out_ref):
    acc = acc_ref[0, :, :] + acc_ref[1, :, :]
    h = jnp.dot(acc, wout_ref[...], preferred_element_type=jnp.float32)
    lr = jnp.dot(ent_ref[...] * lrel_ref[...], wloop_ref[...],
                 preferred_element_type=jnp.float32)
    out = h * 0.5 + lr * 0.5 + bias_ref[...]
    mask = (lax.broadcasted_iota(jnp.int32, (NPAD, 1), 0)
            < N_ENT).astype(jnp.float32)
    om = out * mask
    mean = jnp.sum(om, axis=0, keepdims=True) / N_ENT
    var = jnp.sum(om * om, axis=0, keepdims=True) / N_ENT - mean * mean
    y = (out - mean) * lax.rsqrt(var + 1e-5) * gam_ref[...] + bet_ref[...]
    out_ref[...] = jnp.where(y >= 0.0, y, 0.01 * y)


@functools.cache
def _sc_kernels():
    sc_mesh = plsc.VectorSubcoreMesh(core_axis_name="c", subcore_axis_name="s",
                                     num_cores=NC, num_subcores=NS)
    params = pltpu.CompilerParams(needs_layout_passes=False)
    pass_a = pl.kernel(
        _pass_a_body,
        out_type=(
            jax.ShapeDtypeStruct((NW, EPW), jnp.float32),
            jax.ShapeDtypeStruct((NW, NPAD), jnp.float32),
            jax.ShapeDtypeStruct((NW, NPAD), jnp.float32),
        ),
        mesh=sc_mesh,
        compiler_params=params,
        scratch_types=[
            pltpu.VMEM((EPW,), jnp.int32),
            pltpu.VMEM((EPW,), jnp.int32),
            pltpu.VMEM((EPW,), jnp.int32),
            pltpu.VMEM((EPW,), jnp.float32),
            pltpu.VMEM((2 * RPAD,), jnp.float32),
            pltpu.VMEM((NPAD,), jnp.float32),
            pltpu.VMEM((NPAD,), jnp.float32),
        ],
    )

    pass_c = pl.kernel(
        _pass_c_body,
        out_type=jax.ShapeDtypeStruct((NW, NB2, 4, KB), jnp.int32),
        mesh=sc_mesh,
        compiler_params=params,
        scratch_types=[
            pltpu.VMEM((EPW,), jnp.int32),
            pltpu.VMEM((EPW,), jnp.int32),
            pltpu.VMEM((EPW,), jnp.int32),
            pltpu.VMEM((EPW,), jnp.float32),
            pltpu.VMEM((NPAD,), jnp.float32),
            pltpu.VMEM((NPAD,), jnp.float32),
            pltpu.VMEM((NB2, 4, KB), jnp.int32),
        ],
    )

    pass_b = pl.kernel(
        _pass_b_body,
        out_type=jax.ShapeDtypeStruct((NC, NPAD, D), jnp.float32),
        mesh=sc_mesh,
        compiler_params=params,
        scratch_types=[
            pltpu.VMEM((RTBL * D,), jnp.float32),
            pltpu.VMEM((KB, D), jnp.float32),
            pltpu.VMEM((KB, D), jnp.float32),
            pltpu.VMEM((4, KB), jnp.int32),
            pltpu.VMEM((4, KB), jnp.int32),
            pltpu.VMEM((4, KB), jnp.int32),
            pltpu.VMEM((4, KB), jnp.int32),
            pltpu.VMEM_SHARED((NPAD, D), jnp.float32),
            pltpu.SemaphoreType.DMA,
            pltpu.SemaphoreType.DMA,
            pltpu.SemaphoreType.DMA,
            pltpu.SemaphoreType.DMA,
            pltpu.SemaphoreType.DMA,
            pltpu.SemaphoreType.DMA,
            pltpu.SemaphoreType.DMA,
            pltpu.SemaphoreType.DMA,
        ],
    )
    return pass_a, pass_c, pass_b


def kernel(edge_index, edge_type, query_type, entity_embed, rel_embed,
           w_loop, w_out, w_rel, loop_rel, W_weight, W_bias, a_weight,
           a_bias, bias, bn_gamma, bn_beta):
    f32 = jnp.float32
    row = edge_index[0].astype(jnp.int32)
    col = edge_index[1].astype(jnp.int32)
    et = edge_type.astype(jnp.int32)
    qt = query_type.astype(jnp.int32)

    epad = EPAD - N_EDGE
    row_p = jnp.concatenate([row, jnp.full((epad,), N_ENT, jnp.int32)])
    col_p = jnp.concatenate([col, jnp.zeros((epad,), jnp.int32)])
    et_p = jnp.concatenate([et, jnp.zeros((epad,), jnp.int32)])
    qt_p = jnp.concatenate([qt, jnp.zeros((epad,), jnp.int32)])
    rows2 = row_p.reshape(NW, EPW)
    cols2 = col_p.reshape(NW, EPW)
    et2 = et_p.reshape(NW, EPW)
    qt2 = qt_p.reshape(NW, EPW)

    rel_full = jnp.concatenate([rel_embed, loop_rel], axis=0)
    relf = jnp.pad(rel_full, ((0, RPAD - RTBL), (0, 0)))
    ent_pad = jnp.pad(entity_embed, ((0, NPAD - N_ENT), (0, 0)))

    st, rel_w = pl.pallas_call(
        _k1_body,
        out_shape=(
            jax.ShapeDtypeStruct((2, RPAD), f32),
            jax.ShapeDtypeStruct((RPAD, D), f32),
        ),
    )(relf, W_weight, W_bias.reshape(1, D), a_weight,
      a_bias.reshape(1,), w_rel)

    pass_a, pass_c, pass_b = _sc_kernels()
    alpha2, deg_parts, sa_parts = pass_a(rows2, et2, qt2, st.reshape(-1))

    dinv, g = pl.pallas_call(
        _k3_body,
        out_shape=(
            jax.ShapeDtypeStruct((NPAD // 128, 128), f32),
            jax.ShapeDtypeStruct((NPAD // 128, 128), f32),
        ),
    )(deg_parts.reshape(NW, NPAD // 128, 128),
      sa_parts.reshape(NW, NPAD // 128, 128))

    pk = pass_c(rows2, cols2, et2, alpha2, dinv.reshape(-1), g.reshape(-1))

    acc_parts = pass_b(pk.reshape(NW * NB2, 4, KB), relf[:RTBL].reshape(-1),
                       ent_pad)

    node_out = pl.pallas_call(
        _k5_body,
        out_shape=jax.ShapeDtypeStruct((NPAD, D), f32),
    )(acc_parts, ent_pad, loop_rel, w_out, w_loop, bias.reshape(1, D),
      bn_gamma.reshape(1, D), bn_beta.reshape(1, D))

    return node_out[:N_ENT], rel_w[:N_REL]
